# Initial kernel scaffold; baseline (speedup 1.0000x reference)
import jax
import jax.numpy as jnp
from jax.experimental import pallas as pl


def kernel(x, table):
    raise NotImplementedError("write your pallas kernel here")


# speedup vs baseline: 15.8335x; 15.8335x over previous
import functools

import jax
import jax.numpy as jnp
from jax import lax
from jax.experimental import pallas as pl
from jax.experimental.pallas import tpu as pltpu
from jax.experimental.pallas import tpu_sc as plsc

VOCAB = 100000
DIM = 64
B = 16384
L = 200

NC = 2
NS = 16
NW = NC * NS
ROWS_PER_W = B // NW
C = 4
CHUNKS = ROWS_PER_W // C
HALF = L // 2
NG = 2 * C
NIDX = C * L
LANES = 16
NV = DIM // LANES


def _sum_rows(rows_ref, base):
    zeros = jnp.zeros((LANES,), jnp.float32)

    def body(t, accs):
        r = base + t
        return tuple(
            accs[k] + rows_ref[r, pl.ds(k * LANES, LANES)] for k in range(NV)
        )

    return lax.fori_loop(0, L, body, (zeros,) * NV)


def _worker(xr_hbm, table_hbm, out_hbm, idx_v, rows_v, acc_v, gsem):
    wid = lax.axis_index("s") * NC + lax.axis_index("c")
    base_row = wid * ROWS_PER_W

    def chunk_body(g, carry):
        row0 = base_row + g * C
        pltpu.sync_copy(xr_hbm.at[pl.ds(row0 * 2, NG)], idx_v)
        for j in range(NG):
            pltpu.async_copy(
                table_hbm.at[idx_v.at[j]],
                rows_v.at[pl.ds(j * HALF, HALF)],
                gsem,
            )
        pltpu.make_async_copy(
            table_hbm.at[pl.ds(0, NIDX)], rows_v, gsem
        ).wait()
        for i in range(C):
            accs = _sum_rows(rows_v, i * L)
            for k in range(NV):
                acc_v[i, pl.ds(k * LANES, LANES)] = accs[k]
        pltpu.sync_copy(acc_v, out_hbm.at[pl.ds(row0, C)])
        return carry

    lax.fori_loop(0, CHUNKS, chunk_body, 0)


@jax.jit
def _pooled_lookup(xr, table):
    mesh = plsc.VectorSubcoreMesh(core_axis_name="c", subcore_axis_name="s")
    return pl.kernel(
        _worker,
        mesh=mesh,
        compiler_params=pltpu.CompilerParams(use_tc_tiling_on_sc=False),
        out_type=jax.ShapeDtypeStruct((B, DIM), jnp.float32),
        scratch_types=[
            pltpu.VMEM((NG, HALF), jnp.int32),
            pltpu.VMEM((NIDX, DIM), jnp.float32),
            pltpu.VMEM((C, DIM), jnp.float32),
            pltpu.SemaphoreType.DMA,
        ],
    )(xr, table)


def kernel(x, table):
    xr = x.astype(jnp.int32).reshape(B * 2, HALF)
    return _pooled_lookup(xr, table)


# speedup vs baseline: 31.9030x; 2.0149x over previous
import functools

import jax
import jax.numpy as jnp
from jax import lax
from jax.experimental import pallas as pl
from jax.experimental.pallas import tpu as pltpu
from jax.experimental.pallas import tpu_sc as plsc

VOCAB = 100000
DIM = 64
B = 16384
L = 200

NC = 2
NS = 16
NW = NC * NS
ROWS_PER_W = B // NW
C = 4
CHUNKS = ROWS_PER_W // C
HALF = L // 2
NG = 2 * C
NIDX = C * L
LANES = 16
NV = DIM // LANES
UNROLL = 4


def _sum_rows(rows, base):
    zeros = jnp.zeros((LANES,), jnp.float32)

    def body(t, accs):
        r = base + t * UNROLL
        a = list(accs)
        for u in range(UNROLL):
            for k in range(NV):
                a[k] = a[k] + rows[r + u, pl.ds(k * LANES, LANES)]
        return tuple(a)

    return lax.fori_loop(0, L // UNROLL, body, (zeros,) * NV)


def _worker(xr_hbm, table_hbm, out_hbm, idx_v, rows_v, acc_v,
            isem0, isem1, gsem0, gsem1):
    wid = lax.axis_index("s") * NC + lax.axis_index("c")
    base_row = wid * ROWS_PER_W
    isems = (isem0, isem1)
    gsems = (gsem0, gsem1)

    def idx_start(g, b):
        r0 = (base_row + g * C) * 2
        pltpu.async_copy(xr_hbm.at[pl.ds(r0, NG)], idx_v.at[b], isems[b])

    def idx_wait(b):
        pltpu.make_async_copy(
            xr_hbm.at[pl.ds(0, NG)], idx_v.at[b], isems[b]
        ).wait()

    def gathers_start(b):
        for j in range(NG):
            pltpu.async_copy(
                table_hbm.at[idx_v.at[b, j]],
                rows_v.at[b, pl.ds(j * HALF, HALF)],
                gsems[b],
            )

    def gathers_drain(b):
        pltpu.make_async_copy(
            table_hbm.at[pl.ds(0, NIDX)], rows_v.at[b], gsems[b]
        ).wait()

    def consume(g, b):
        for i in range(C):
            accs = _sum_rows(rows_v.at[b], i * L)
            for k in range(NV):
                acc_v[i, pl.ds(k * LANES, LANES)] = accs[k]
        pltpu.sync_copy(acc_v, out_hbm.at[pl.ds(base_row + g * C, C)])

    idx_start(0, 0)
    idx_start(1, 1)
    idx_wait(0)
    gathers_start(0)

    def pair_body(h, carry):
        g = 2 * h
        idx_wait(1)
        gathers_start(1)
        gathers_drain(0)
        idx_start(g + 2, 0)
        consume(g, 0)
        idx_wait(0)
        gathers_start(0)
        gathers_drain(1)
        idx_start(g + 3, 1)
        consume(g + 1, 1)
        return carry

    lax.fori_loop(0, CHUNKS // 2 - 1, pair_body, 0)

    idx_wait(1)
    gathers_start(1)
    gathers_drain(0)
    consume(CHUNKS - 2, 0)
    gathers_drain(1)
    consume(CHUNKS - 1, 1)


@jax.jit
def _pooled_lookup(xr, table):
    mesh = plsc.VectorSubcoreMesh(core_axis_name="c", subcore_axis_name="s")
    return pl.kernel(
        _worker,
        mesh=mesh,
        compiler_params=pltpu.CompilerParams(use_tc_tiling_on_sc=False),
        out_type=jax.ShapeDtypeStruct((B, DIM), jnp.float32),
        scratch_types=[
            pltpu.VMEM((2, NG, HALF), jnp.int32),
            pltpu.VMEM((2, NIDX, DIM), jnp.float32),
            pltpu.VMEM((C, DIM), jnp.float32),
            pltpu.SemaphoreType.DMA,
            pltpu.SemaphoreType.DMA,
            pltpu.SemaphoreType.DMA,
            pltpu.SemaphoreType.DMA,
        ],
    )(xr, table)


def kernel(x, table):
    xr = x.astype(jnp.int32).reshape(B * 2, HALF)
    return _pooled_lookup(xr, table)


# speedup vs baseline: 38.1862x; 1.1969x over previous
import functools

import numpy as np
import jax
import jax.numpy as jnp
from jax import lax
from jax.experimental import pallas as pl
from jax.experimental.pallas import tpu as pltpu
from jax.experimental.pallas import tpu_sc as plsc

VOCAB = 100000
DIM = 64
B = 16384
L = 200

NC = 2
NS = 16
NW = NC * NS
ROWS_PER_W = B // NW
C = 8
CHUNKS = ROWS_PER_W // C
HALF = L // 2
NG = 2 * C
NIDX = C * L
LANES = 16
NG32 = DIM // 32
UNROLL = 2

_PERM = np.asarray(
    [k * 32 + (j // 2) + 16 * (j % 2) for k in range(NG32) for j in range(32)],
    dtype=np.int32,
)


def _sum_rows(rows, base):
    zeros = jnp.zeros((LANES,), jnp.float32)

    def body(t, accs):
        r = base + t * UNROLL
        a = list(accs)
        for u in range(UNROLL):
            for k in range(NG32):
                packed = rows[r + u, pl.ds(k * 32, 32)]
                lo, hi = plsc.unpack(packed, format=plsc.PackFormat.INTERLEAVED)
                a[2 * k] = a[2 * k] + lo
                a[2 * k + 1] = a[2 * k + 1] + hi
        return tuple(a)

    return lax.fori_loop(0, L // UNROLL, body, (zeros,) * 4)


def _worker(xr_hbm, table_hbm, out_hbm, idx_v, rows_v, acc_v,
            isem0, isem1, gsem0, gsem1):
    wid = lax.axis_index("s") * NC + lax.axis_index("c")
    base_row = wid * ROWS_PER_W
    isems = (isem0, isem1)
    gsems = (gsem0, gsem1)

    def idx_start(g, b):
        r0 = (base_row + g * C) * 2
        pltpu.async_copy(xr_hbm.at[pl.ds(r0, NG)], idx_v.at[b], isems[b])

    def idx_wait(b):
        pltpu.make_async_copy(
            xr_hbm.at[pl.ds(0, NG)], idx_v.at[b], isems[b]
        ).wait()

    def gathers_start(b):
        for j in range(NG):
            pltpu.async_copy(
                table_hbm.at[idx_v.at[b, j]],
                rows_v.at[b, pl.ds(j * HALF, HALF)],
                gsems[b],
            )

    def gathers_drain(b):
        pltpu.make_async_copy(
            table_hbm.at[pl.ds(0, NIDX)], rows_v.at[b], gsems[b]
        ).wait()

    def consume(g, b):
        for i in range(C):
            accs = _sum_rows(rows_v.at[b], i * L)
            for k in range(4):
                acc_v[i, pl.ds(k * LANES, LANES)] = accs[k]
        pltpu.sync_copy(acc_v, out_hbm.at[pl.ds(base_row + g * C, C)])

    idx_start(0, 0)
    idx_start(1, 1)
    idx_wait(0)
    gathers_start(0)

    def pair_body(h, carry):
        g = 2 * h
        idx_wait(1)
        gathers_start(1)
        gathers_drain(0)
        idx_start(g + 2, 0)
        consume(g, 0)
        idx_wait(0)
        gathers_start(0)
        gathers_drain(1)
        idx_start(g + 3, 1)
        consume(g + 1, 1)
        return carry

    lax.fori_loop(0, CHUNKS // 2 - 1, pair_body, 0)

    idx_wait(1)
    gathers_start(1)
    gathers_drain(0)
    consume(CHUNKS - 2, 0)
    gathers_drain(1)
    consume(CHUNKS - 1, 1)


@jax.jit
def _pooled_lookup(x, table):
    xr = x.astype(jnp.int32).reshape(B * 2, HALF)
    table_bf16 = table[:, _PERM].astype(jnp.bfloat16)
    mesh = plsc.VectorSubcoreMesh(core_axis_name="c", subcore_axis_name="s")
    return pl.kernel(
        _worker,
        mesh=mesh,
        compiler_params=pltpu.CompilerParams(
            use_tc_tiling_on_sc=False, needs_layout_passes=False
        ),
        out_type=jax.ShapeDtypeStruct((B, DIM), jnp.float32),
        scratch_types=[
            pltpu.VMEM((2, NG, HALF), jnp.int32),
            pltpu.VMEM((2, NIDX, DIM), jnp.bfloat16),
            pltpu.VMEM((C, DIM), jnp.float32),
            pltpu.SemaphoreType.DMA,
            pltpu.SemaphoreType.DMA,
            pltpu.SemaphoreType.DMA,
            pltpu.SemaphoreType.DMA,
        ],
    )(xr, table_bf16)


def kernel(x, table):
    return _pooled_lookup(x, table)


# speedup vs baseline: 43.3310x; 1.1347x over previous
import functools

import jax
import jax.numpy as jnp
from jax import lax
from jax.experimental import pallas as pl
from jax.experimental.pallas import tpu as pltpu
from jax.experimental.pallas import tpu_sc as plsc

VOCAB = 100000
DIM = 64
B = 16384
L = 200

NC = 2
NS = 16
NW = NC * NS
ROWS_PER_W = B // NW
C = 8
CHUNKS = ROWS_PER_W // C
SPLIT = (104, 96)
NG = 2 * C
NIDX = C * L
LANES = 16
NG32 = DIM // 32
UNROLL = 2


def _sum_rows(rows, base):
    zeros = jnp.zeros((LANES,), jnp.float32)

    def body(t, accs):
        r = base + t * UNROLL
        a = list(accs)
        for u in range(UNROLL):
            for k in range(NG32):
                packed = rows[r + u, pl.ds(k * 32, 32)]
                lo, hi = plsc.unpack(packed, format=plsc.PackFormat.INTERLEAVED)
                a[2 * k] = a[2 * k] + lo
                a[2 * k + 1] = a[2 * k + 1] + hi
        return tuple(a)

    return lax.fori_loop(0, L // UNROLL, body, (zeros,) * 4)


def _worker(x_hbm, table_hbm, out_hbm, idx_v, rows_v, acc_v,
            isem0, isem1, gsem0, gsem1):
    wid = lax.axis_index("s") * NC + lax.axis_index("c")
    base_row = wid * ROWS_PER_W
    isems = (isem0, isem1)
    gsems = (gsem0, gsem1)
    even = 2 * lax.iota(jnp.int32, LANES)
    odd = even + 1

    def idx_start(g, b):
        r0 = base_row + g * C
        pltpu.async_copy(x_hbm.at[pl.ds(r0, C)], idx_v.at[b], isems[b])

    def idx_wait(b):
        pltpu.make_async_copy(
            x_hbm.at[pl.ds(0, C)], idx_v.at[b], isems[b]
        ).wait()

    def gathers_start(b):
        for i in range(C):
            off = 0
            for n in SPLIT:
                pltpu.async_copy(
                    table_hbm.at[idx_v.at[b, i, pl.ds(off, n)]],
                    rows_v.at[b, pl.ds(i * L + off, n)],
                    gsems[b],
                )
                off += n

    def gathers_drain(b):
        pltpu.make_async_copy(
            table_hbm.at[pl.ds(0, NIDX)], rows_v.at[b], gsems[b]
        ).wait()

    def consume(g, b):
        for i in range(C):
            accs = _sum_rows(rows_v.at[b], i * L)
            row = jnp.full((LANES,), i, jnp.int32)
            for k in range(NG32):
                plsc.store_scatter(acc_v, [row, k * 32 + even], accs[2 * k])
                plsc.store_scatter(acc_v, [row, k * 32 + odd], accs[2 * k + 1])
        pltpu.sync_copy(acc_v, out_hbm.at[pl.ds(base_row + g * C, C)])

    idx_start(0, 0)
    idx_start(1, 1)
    idx_wait(0)
    gathers_start(0)

    def pair_body(h, carry):
        g = 2 * h
        idx_wait(1)
        gathers_start(1)
        gathers_drain(0)
        idx_start(g + 2, 0)
        consume(g, 0)
        idx_wait(0)
        gathers_start(0)
        gathers_drain(1)
        idx_start(g + 3, 1)
        consume(g + 1, 1)
        return carry

    lax.fori_loop(0, CHUNKS // 2 - 1, pair_body, 0)

    idx_wait(1)
    gathers_start(1)
    gathers_drain(0)
    consume(CHUNKS - 2, 0)
    gathers_drain(1)
    consume(CHUNKS - 1, 1)


@jax.jit
def _pooled_lookup(x, table):
    table_bf16 = table.astype(jnp.bfloat16)
    mesh = plsc.VectorSubcoreMesh(core_axis_name="c", subcore_axis_name="s")
    return pl.kernel(
        _worker,
        mesh=mesh,
        compiler_params=pltpu.CompilerParams(
            use_tc_tiling_on_sc=False, needs_layout_passes=False
        ),
        out_type=jax.ShapeDtypeStruct((B, DIM), jnp.float32),
        scratch_types=[
            pltpu.VMEM((2, C, L), jnp.int32),
            pltpu.VMEM((2, NIDX, DIM), jnp.bfloat16),
            pltpu.VMEM((C, DIM), jnp.float32),
            pltpu.SemaphoreType.DMA,
            pltpu.SemaphoreType.DMA,
            pltpu.SemaphoreType.DMA,
            pltpu.SemaphoreType.DMA,
        ],
    )(x, table_bf16)


def kernel(x, table):
    return _pooled_lookup(x, table)
